# Initial kernel scaffold; baseline (speedup 1.0000x reference)
import jax
import jax.numpy as jnp
from jax.experimental import pallas as pl


def kernel(team_a_ids, team_b_ids, team_a_numerical, team_b_numerical, team_a_class_ids, team_b_class_ids, team_a_damage_one_hot, team_b_damage_one_hot, emb_table, class_table, W1, b1, W2, b2, W3, b3):
    raise NotImplementedError("write your pallas kernel here")


# speedup vs baseline: 5.4043x; 5.4043x over previous
import functools

import jax
import jax.numpy as jnp
from jax import lax
from jax.experimental import pallas as pl
from jax.experimental.pallas import tpu as pltpu
from jax.experimental.pallas import tpu_sc as plsc

B = 16384
L = 5
EMB_DIM = 64
CLASS_DIM = 16
NUM_CORES = 2
NUM_SUBCORES = 16
NW = NUM_CORES * NUM_SUBCORES
IPW = B // NW
CH = 128
NCH = IPW // CH
INV_L = 0.2


def _sc_pool_body(emb_hbm, ctab_hbm, ida_hbm, cida_hbm, idb_hbm, cidb_hbm,
                  ea_hbm, ca_hbm, eb_hbm, cb_hbm,
                  idx_v, cidx_v, bufe, bufc, oute, outc, sem):
  wid = lax.axis_index("s") * NUM_CORES + lax.axis_index("c")
  base = wid * IPW

  for (id3, cid3, eout, cout) in ((ida_hbm, cida_hbm, ea_hbm, ca_hbm),
                                  (idb_hbm, cidb_hbm, eb_hbm, cb_hbm)):
    pltpu.sync_copy(id3.at[:, pl.ds(wid * NCH, NCH), :], idx_v)
    pltpu.sync_copy(cid3.at[:, pl.ds(wid * NCH, NCH), :], cidx_v)
    for cc in range(NCH):
      cps = []
      for k in range(L):
        cps.append(pltpu.async_copy(emb_hbm.at[idx_v.at[k, cc]],
                                    bufe.at[k], sem))
        cps.append(pltpu.async_copy(ctab_hbm.at[cidx_v.at[k, cc]],
                                    bufc.at[k], sem))
      for cp in cps:
        cp.wait()

      def red_body(c, carry):
        for j in range(EMB_DIM // 16):
          s = bufe[0, c, pl.ds(16 * j, 16)]
          for k in range(1, L):
            s = s + bufe[k, c, pl.ds(16 * j, 16)]
          oute[c, pl.ds(16 * j, 16)] = s * INV_L
        sc = bufc[0, c, :]
        for k in range(1, L):
          sc = sc + bufc[k, c, :]
        outc[c, :] = sc * INV_L
        return carry

      lax.fori_loop(0, CH, red_body, 0, unroll=False)
      pltpu.sync_copy(oute, eout.at[pl.ds(base + cc * CH, CH)])
      pltpu.sync_copy(outc, cout.at[pl.ds(base + cc * CH, CH)])


def _sc_pool(emb_table, class_table, ida3, cida3, idb3, cidb3):
  mesh = plsc.VectorSubcoreMesh(core_axis_name="c", subcore_axis_name="s")
  out_type = (jax.ShapeDtypeStruct((B, EMB_DIM), jnp.float32),
              jax.ShapeDtypeStruct((B, CLASS_DIM), jnp.float32),
              jax.ShapeDtypeStruct((B, EMB_DIM), jnp.float32),
              jax.ShapeDtypeStruct((B, CLASS_DIM), jnp.float32))
  scratch = [
      pltpu.VMEM((L, NCH, CH), jnp.int32),
      pltpu.VMEM((L, NCH, CH), jnp.int32),
      pltpu.VMEM((L, CH, EMB_DIM), jnp.float32),
      pltpu.VMEM((L, CH, CLASS_DIM), jnp.float32),
      pltpu.VMEM((CH, EMB_DIM), jnp.float32),
      pltpu.VMEM((CH, CLASS_DIM), jnp.float32),
      pltpu.SemaphoreType.DMA,
  ]
  fn = pl.kernel(_sc_pool_body, out_type=out_type, mesh=mesh,
                 scratch_types=scratch,
                 compiler_params=pltpu.CompilerParams(
                     use_tc_tiling_on_sc=False))
  return fn(emb_table, class_table, ida3, cida3, idb3, cidb3)


BM = 512


def _tc_mlp_body(ea, ca, eb, cb, na_r, da_r, nb_r, db_r,
                 w1ea, w1na, w1ca, w1da, w1eb, w1nb, w1cb, w1db,
                 b1, w2, b2, w3, b3, out):
  def pool5(ref, width):
    s = ref[:, pl.ds(0, width)]
    for k in range(1, L):
      s = s + ref[:, pl.ds(k * width, width)]
    return s * INV_L

  na = pool5(na_r, 32)
  nb = pool5(nb_r, 32)
  da = pool5(da_r, 3)
  db = pool5(db_r, 3)
  zpad = jnp.zeros((BM, 5), dtype=jnp.float32)
  da8 = jnp.concatenate([da, zpad], axis=1)
  db8 = jnp.concatenate([db, zpad], axis=1)

  h = jnp.dot(ea[...], w1ea[...], preferred_element_type=jnp.float32)
  h = h + jnp.dot(na, w1na[...], preferred_element_type=jnp.float32)
  h = h + jnp.dot(ca[...], w1ca[...], preferred_element_type=jnp.float32)
  h = h + jnp.dot(da8, w1da[...], preferred_element_type=jnp.float32)
  h = h + jnp.dot(eb[...], w1eb[...], preferred_element_type=jnp.float32)
  h = h + jnp.dot(nb, w1nb[...], preferred_element_type=jnp.float32)
  h = h + jnp.dot(cb[...], w1cb[...], preferred_element_type=jnp.float32)
  h = h + jnp.dot(db8, w1db[...], preferred_element_type=jnp.float32)
  h = jnp.maximum(h + b1[...], 0.0)
  h2 = jnp.dot(h, w2[...], preferred_element_type=jnp.float32)
  h2 = jnp.maximum(h2 + b2[...], 0.0)
  o = jnp.dot(h2, w3[...], preferred_element_type=jnp.float32)
  out[...] = jax.nn.sigmoid(o + b3[...])


def _tc_mlp(ea, ca, eb, cb, na_r, da_r, nb_r, db_r, w1s, b1, w2, b2, w3, b3):
  grid = (B // BM,)
  row = lambda i: (i, 0)
  const = lambda i: (0, 0)
  in_specs = [
      pl.BlockSpec((BM, EMB_DIM), row),
      pl.BlockSpec((BM, CLASS_DIM), row),
      pl.BlockSpec((BM, EMB_DIM), row),
      pl.BlockSpec((BM, CLASS_DIM), row),
      pl.BlockSpec((BM, 160), row),
      pl.BlockSpec((BM, 15), row),
      pl.BlockSpec((BM, 160), row),
      pl.BlockSpec((BM, 15), row),
  ]
  for w in w1s:
    in_specs.append(pl.BlockSpec(w.shape, const))
  in_specs += [
      pl.BlockSpec((1, 512), const),
      pl.BlockSpec((512, 256), const),
      pl.BlockSpec((1, 256), const),
      pl.BlockSpec((256, 1), const),
      pl.BlockSpec((1, 1), const),
  ]
  out = pl.pallas_call(
      _tc_mlp_body,
      grid=grid,
      in_specs=in_specs,
      out_specs=pl.BlockSpec((BM, 1), row),
      out_shape=jax.ShapeDtypeStruct((B, 1), jnp.float32),
      compiler_params=pltpu.CompilerParams(
          dimension_semantics=("parallel",)),
  )(ea, ca, eb, cb, na_r, da_r, nb_r, db_r, *w1s, b1, w2, b2, w3, b3)
  return out


def kernel(team_a_ids, team_b_ids, team_a_numerical, team_b_numerical,
           team_a_class_ids, team_b_class_ids, team_a_damage_one_hot,
           team_b_damage_one_hot, emb_table, class_table, W1, b1, W2, b2,
           W3, b3):
  ids3 = lambda ids: ids.astype(jnp.int32).T.reshape(L, B // CH, CH)
  ea, ca, eb, cb = _sc_pool(emb_table, class_table,
                            ids3(team_a_ids), ids3(team_a_class_ids),
                            ids3(team_b_ids), ids3(team_b_class_ids))

  na_r = team_a_numerical.reshape(B, L * 32)
  nb_r = team_b_numerical.reshape(B, L * 32)
  da_r = team_a_damage_one_hot.reshape(B, L * 3)
  db_r = team_b_damage_one_hot.reshape(B, L * 3)

  pad5 = jnp.zeros((5, 512), dtype=jnp.float32)
  w1s = (W1[0:64], W1[64:96], W1[96:112],
         jnp.concatenate([W1[112:115], pad5], axis=0),
         W1[115:179], W1[179:211], W1[211:227],
         jnp.concatenate([W1[227:230], pad5], axis=0))

  out = _tc_mlp(ea, ca, eb, cb, na_r, da_r, nb_r, db_r, w1s,
                b1.reshape(1, 512), W2, b2.reshape(1, 256), W3,
                b3.reshape(1, 1))
  return out.reshape(B)


# speedup vs baseline: 5.6287x; 1.0415x over previous
import functools

import jax
import jax.numpy as jnp
from jax import lax
from jax.experimental import pallas as pl
from jax.experimental.pallas import tpu as pltpu
from jax.experimental.pallas import tpu_sc as plsc

B = 16384
L = 5
EMB_DIM = 64
CLASS_DIM = 16
NUM_CORES = 2
NUM_SUBCORES = 16
NW = NUM_CORES * NUM_SUBCORES
IPW = B // NW
CH = 128
NCH = IPW // CH
NSTEP = 2 * NCH
INV_L = 0.2


def _sc_pool_body(emb_hbm, ctab_hbm, ida_hbm, cida_hbm, idb_hbm, cidb_hbm,
                  ea_hbm, ca_hbm, eb_hbm, cb_hbm,
                  idxa, cidxa, idxb, cidxb,
                  bufe0, bufc0, bufe1, bufc1, oute, outc,
                  sem0, sem1, semw):
  wid = lax.axis_index("s") * NUM_CORES + lax.axis_index("c")
  base = wid * IPW

  pltpu.sync_copy(ida_hbm.at[:, pl.ds(wid * NCH, NCH), :], idxa)
  pltpu.sync_copy(cida_hbm.at[:, pl.ds(wid * NCH, NCH), :], cidxa)
  pltpu.sync_copy(idb_hbm.at[:, pl.ds(wid * NCH, NCH), :], idxb)
  pltpu.sync_copy(cidb_hbm.at[:, pl.ds(wid * NCH, NCH), :], cidxb)

  steps = ([(idxa, cidxa, ea_hbm, ca_hbm, cc) for cc in range(NCH)]
           + [(idxb, cidxb, eb_hbm, cb_hbm, cc) for cc in range(NCH)])
  bufs = [(bufe0, bufc0, sem0), (bufe1, bufc1, sem1)]

  def fire(s):
    idx, cidx, _, _, cc = steps[s]
    be, bc, sem = bufs[s % 2]
    cps = []
    for k in range(L):
      cps.append(pltpu.async_copy(emb_hbm.at[idx.at[k, cc]], be.at[k], sem))
      cps.append(pltpu.async_copy(ctab_hbm.at[cidx.at[k, cc]], bc.at[k],
                                  sem))
    return cps

  pend = fire(0)
  wpend = []
  for s in range(NSTEP):
    nxt = fire(s + 1) if s + 1 < NSTEP else []
    for cp in pend:
      cp.wait()
    pend = nxt
    for cp in wpend:
      cp.wait()
    be, bc, _ = bufs[s % 2]
    _, _, eout, cout, cc = steps[s]

    def red_body(c, carry, be=be, bc=bc):
      for j in range(EMB_DIM // 16):
        acc = be[0, c, pl.ds(16 * j, 16)]
        for k in range(1, L):
          acc = acc + be[k, c, pl.ds(16 * j, 16)]
        oute[c, pl.ds(16 * j, 16)] = acc
      accc = bc[0, c, :]
      for k in range(1, L):
        accc = accc + bc[k, c, :]
      outc[c, :] = accc
      return carry

    lax.fori_loop(0, CH, red_body, 0, unroll=False)
    wpend = [
        pltpu.async_copy(oute, eout.at[pl.ds(base + cc * CH, CH)], semw),
        pltpu.async_copy(outc, cout.at[pl.ds(base + cc * CH, CH)], semw),
    ]
  for cp in wpend:
    cp.wait()


def _sc_pool(emb_table, class_table, ida3, cida3, idb3, cidb3):
  mesh = plsc.VectorSubcoreMesh(core_axis_name="c", subcore_axis_name="s")
  out_type = (jax.ShapeDtypeStruct((B, EMB_DIM), jnp.float32),
              jax.ShapeDtypeStruct((B, CLASS_DIM), jnp.float32),
              jax.ShapeDtypeStruct((B, EMB_DIM), jnp.float32),
              jax.ShapeDtypeStruct((B, CLASS_DIM), jnp.float32))
  scratch = [
      pltpu.VMEM((L, NCH, CH), jnp.int32),
      pltpu.VMEM((L, NCH, CH), jnp.int32),
      pltpu.VMEM((L, NCH, CH), jnp.int32),
      pltpu.VMEM((L, NCH, CH), jnp.int32),
      pltpu.VMEM((L, CH, EMB_DIM), jnp.float32),
      pltpu.VMEM((L, CH, CLASS_DIM), jnp.float32),
      pltpu.VMEM((L, CH, EMB_DIM), jnp.float32),
      pltpu.VMEM((L, CH, CLASS_DIM), jnp.float32),
      pltpu.VMEM((CH, EMB_DIM), jnp.float32),
      pltpu.VMEM((CH, CLASS_DIM), jnp.float32),
      pltpu.SemaphoreType.DMA,
      pltpu.SemaphoreType.DMA,
      pltpu.SemaphoreType.DMA,
  ]
  fn = pl.kernel(_sc_pool_body, out_type=out_type, mesh=mesh,
                 scratch_types=scratch,
                 compiler_params=pltpu.CompilerParams(
                     use_tc_tiling_on_sc=False))
  return fn(emb_table, class_table, ida3, cida3, idb3, cidb3)


BM = 512
BF = jnp.bfloat16


def _tc_mlp_body(ea, ca, eb, cb, na_r, da_r, nb_r, db_r,
                 w1ea, w1na, w1ca, w1da, w1eb, w1nb, w1cb, w1db,
                 b1, w2, b2, w3, b3, out):
  def pool5(ref, width):
    s = ref[:, pl.ds(0, width)]
    for k in range(1, L):
      s = s + ref[:, pl.ds(k * width, width)]
    return (s * INV_L).astype(BF)

  na = pool5(na_r, 32)
  nb = pool5(nb_r, 32)
  da = pool5(da_r, 3)
  db = pool5(db_r, 3)
  zpad = jnp.zeros((BM, 5), dtype=BF)
  da8 = jnp.concatenate([da, zpad], axis=1)
  db8 = jnp.concatenate([db, zpad], axis=1)

  f32 = jnp.float32
  h = jnp.dot(ea[...].astype(BF), w1ea[...], preferred_element_type=f32)
  h = h + jnp.dot(na, w1na[...], preferred_element_type=f32)
  h = h + jnp.dot(ca[...].astype(BF), w1ca[...], preferred_element_type=f32)
  h = h + jnp.dot(da8, w1da[...], preferred_element_type=f32)
  h = h + jnp.dot(eb[...].astype(BF), w1eb[...], preferred_element_type=f32)
  h = h + jnp.dot(nb, w1nb[...], preferred_element_type=f32)
  h = h + jnp.dot(cb[...].astype(BF), w1cb[...], preferred_element_type=f32)
  h = h + jnp.dot(db8, w1db[...], preferred_element_type=f32)
  h = jnp.maximum(h + b1[...], 0.0).astype(BF)
  h2 = jnp.dot(h, w2[...], preferred_element_type=f32)
  h2 = jnp.maximum(h2 + b2[...], 0.0).astype(BF)
  o = jnp.dot(h2, w3[...], preferred_element_type=f32)
  out[...] = jax.nn.sigmoid(o + b3[...])


def _tc_mlp(ea, ca, eb, cb, na_r, da_r, nb_r, db_r, w1s, b1, w2, b2, w3, b3):
  grid = (B // BM,)
  row = lambda i: (i, 0)
  const = lambda i: (0, 0)
  in_specs = [
      pl.BlockSpec((BM, EMB_DIM), row),
      pl.BlockSpec((BM, CLASS_DIM), row),
      pl.BlockSpec((BM, EMB_DIM), row),
      pl.BlockSpec((BM, CLASS_DIM), row),
      pl.BlockSpec((BM, 160), row),
      pl.BlockSpec((BM, 15), row),
      pl.BlockSpec((BM, 160), row),
      pl.BlockSpec((BM, 15), row),
  ]
  for w in w1s:
    in_specs.append(pl.BlockSpec(w.shape, const))
  in_specs += [
      pl.BlockSpec((1, 512), const),
      pl.BlockSpec((512, 256), const),
      pl.BlockSpec((1, 256), const),
      pl.BlockSpec((256, 1), const),
      pl.BlockSpec((1, 1), const),
  ]
  out = pl.pallas_call(
      _tc_mlp_body,
      grid=grid,
      in_specs=in_specs,
      out_specs=pl.BlockSpec((BM, 1), row),
      out_shape=jax.ShapeDtypeStruct((B, 1), jnp.float32),
      compiler_params=pltpu.CompilerParams(
          dimension_semantics=("parallel",)),
  )(ea, ca, eb, cb, na_r, da_r, nb_r, db_r, *w1s, b1, w2, b2, w3, b3)
  return out


def kernel(team_a_ids, team_b_ids, team_a_numerical, team_b_numerical,
           team_a_class_ids, team_b_class_ids, team_a_damage_one_hot,
           team_b_damage_one_hot, emb_table, class_table, W1, b1, W2, b2,
           W3, b3):
  ids3 = lambda ids: ids.astype(jnp.int32).T.reshape(L, B // CH, CH)
  ea, ca, eb, cb = _sc_pool(emb_table, class_table,
                            ids3(team_a_ids), ids3(team_a_class_ids),
                            ids3(team_b_ids), ids3(team_b_class_ids))

  na_r = team_a_numerical.reshape(B, L * 32)
  nb_r = team_b_numerical.reshape(B, L * 32)
  da_r = team_a_damage_one_hot.reshape(B, L * 3)
  db_r = team_b_damage_one_hot.reshape(B, L * 3)

  pad5 = jnp.zeros((5, 512), dtype=jnp.float32)
  w1s = tuple(w.astype(BF) for w in (
      W1[0:64] * INV_L, W1[64:96], W1[96:112] * INV_L,
      jnp.concatenate([W1[112:115], pad5], axis=0),
      W1[115:179] * INV_L, W1[179:211], W1[211:227] * INV_L,
      jnp.concatenate([W1[227:230], pad5], axis=0)))

  out = _tc_mlp(ea, ca, eb, cb, na_r, da_r, nb_r, db_r, w1s,
                b1.reshape(1, 512), W2.astype(BF), b2.reshape(1, 256),
                W3.astype(BF), b3.reshape(1, 1))
  return out.reshape(B)


# speedup vs baseline: 6.0317x; 1.0716x over previous
import functools

import jax
import jax.numpy as jnp
from jax import lax
from jax.experimental import pallas as pl
from jax.experimental.pallas import tpu as pltpu
from jax.experimental.pallas import tpu_sc as plsc

B = 16384
L = 5
EMB_DIM = 64
CLASS_DIM = 16
FEAT = 80
FPAD = 128
ND = 175
NUM_CORES = 2
NUM_SUBCORES = 16
NW = NUM_CORES * NUM_SUBCORES
IPW = B // NW
CH = 128
NCH = IPW // CH
NSTEP = 2 * NCH
INV_L = 0.2


def _sc_pool_body(emb_hbm, ctab_hbm, ida_hbm, cida_hbm, idb_hbm, cidb_hbm,
                  outa_hbm, outb_hbm,
                  idxe_a, idxc_a, idxe_b, idxc_b,
                  bufe0, bufc0, bufe1, bufc1, oute,
                  sem0, sem1, semw):
  wid = lax.axis_index("s") * NUM_CORES + lax.axis_index("c")
  base = wid * IPW

  for src, dst in ((ida_hbm, idxe_a), (cida_hbm, idxc_a),
                   (idb_hbm, idxe_b), (cidb_hbm, idxc_b)):
    for k in range(L):
      pltpu.sync_copy(src.at[k, pl.ds(base, IPW)], dst.at[k])

  steps = ([(idxe_a, idxc_a, outa_hbm, cc) for cc in range(NCH)]
           + [(idxe_b, idxc_b, outb_hbm, cc) for cc in range(NCH)])
  bufs = [(bufe0, bufc0, sem0), (bufe1, bufc1, sem1)]

  def fire(s):
    idx, cidx, _, cc = steps[s]
    be, bc, sem = bufs[s % 2]
    cps = []
    for k in range(L):
      cps.append(pltpu.async_copy(
          emb_hbm.at[idx.at[k, pl.ds(cc * CH, CH)]], be.at[k], sem))
      cps.append(pltpu.async_copy(
          ctab_hbm.at[cidx.at[k, pl.ds(cc * CH, CH)]], bc.at[k], sem))
    return cps

  pend = fire(0)
  wpend = []
  for s in range(NSTEP):
    nxt = fire(s + 1) if s + 1 < NSTEP else []
    for cp in pend:
      cp.wait()
    pend = nxt
    for cp in wpend:
      cp.wait()
    be, bc, _ = bufs[s % 2]
    _, _, out_hbm, cc = steps[s]

    def red_body(c, carry, be=be, bc=bc):
      for j in range(EMB_DIM // 16):
        acc = be[0, c, pl.ds(16 * j, 16)]
        for k in range(1, L):
          acc = acc + be[k, c, pl.ds(16 * j, 16)]
        oute[c, pl.ds(16 * j, 16)] = acc
      accc = bc[0, c, :]
      for k in range(1, L):
        accc = accc + bc[k, c, :]
      oute[c, pl.ds(EMB_DIM, CLASS_DIM)] = accc
      return carry

    lax.fori_loop(0, CH, red_body, 0, unroll=False)
    wpend = [
        pltpu.async_copy(
            oute, out_hbm.at[pl.ds(base + cc * CH, CH), pl.ds(0, FEAT)],
            semw),
    ]
  for cp in wpend:
    cp.wait()


def _sc_pool(emb_table, class_table, ida, cida, idb, cidb):
  mesh = plsc.VectorSubcoreMesh(core_axis_name="c", subcore_axis_name="s")
  out_type = (jax.ShapeDtypeStruct((B, FPAD), jnp.float32),
              jax.ShapeDtypeStruct((B, FPAD), jnp.float32))
  scratch = [
      pltpu.VMEM((L, IPW), jnp.int32),
      pltpu.VMEM((L, IPW), jnp.int32),
      pltpu.VMEM((L, IPW), jnp.int32),
      pltpu.VMEM((L, IPW), jnp.int32),
      pltpu.VMEM((L, CH, EMB_DIM), jnp.float32),
      pltpu.VMEM((L, CH, CLASS_DIM), jnp.float32),
      pltpu.VMEM((L, CH, EMB_DIM), jnp.float32),
      pltpu.VMEM((L, CH, CLASS_DIM), jnp.float32),
      pltpu.VMEM((CH, FEAT), jnp.float32),
      pltpu.SemaphoreType.DMA,
      pltpu.SemaphoreType.DMA,
      pltpu.SemaphoreType.DMA,
  ]
  fn = pl.kernel(_sc_pool_body, out_type=out_type, mesh=mesh,
                 scratch_types=scratch,
                 compiler_params=pltpu.CompilerParams(
                     use_tc_tiling_on_sc=False))
  return fn(emb_table, class_table, ida, cida, idb, cidb)


BM = 1024
BF = jnp.bfloat16


def _tc_mlp_body(fa, fb, nd_a, nd_b,
                 w1fa, w1fb, wnd_a, wnd_b, b1, w2, b2, w3, b3, out):
  f32 = jnp.float32
  xa = fa[:, pl.ds(0, FEAT)].astype(BF)
  xb = fb[:, pl.ds(0, FEAT)].astype(BF)
  na = nd_a[...].astype(BF)
  nb = nd_b[...].astype(BF)
  h = jnp.dot(xa, w1fa[...], preferred_element_type=f32)
  h = h + jnp.dot(na, wnd_a[...], preferred_element_type=f32)
  h = h + jnp.dot(xb, w1fb[...], preferred_element_type=f32)
  h = h + jnp.dot(nb, wnd_b[...], preferred_element_type=f32)
  h = jnp.maximum(h + b1[...], 0.0).astype(BF)
  h2 = jnp.dot(h, w2[...], preferred_element_type=f32)
  h2 = jnp.maximum(h2 + b2[...], 0.0).astype(BF)
  o = jnp.dot(h2, w3[...], preferred_element_type=f32)
  out[...] = jax.nn.sigmoid(o + b3[...])


def _tc_mlp(fa, fb, nd_a, nd_b, w1fa, w1fb, wnd_a, wnd_b, b1, w2, b2, w3,
            b3):
  grid = (B // BM,)
  row = lambda i: (i, 0)
  const = lambda i: (0, 0)
  in_specs = [
      pl.BlockSpec((BM, FPAD), row),
      pl.BlockSpec((BM, FPAD), row),
      pl.BlockSpec((BM, ND), row),
      pl.BlockSpec((BM, ND), row),
      pl.BlockSpec((FEAT, 512), const),
      pl.BlockSpec((FEAT, 512), const),
      pl.BlockSpec((ND, 512), const),
      pl.BlockSpec((ND, 512), const),
      pl.BlockSpec((1, 512), const),
      pl.BlockSpec((512, 256), const),
      pl.BlockSpec((1, 256), const),
      pl.BlockSpec((256, 1), const),
      pl.BlockSpec((1, 1), const),
  ]
  out = pl.pallas_call(
      _tc_mlp_body,
      grid=grid,
      in_specs=in_specs,
      out_specs=pl.BlockSpec((BM, 1), row),
      out_shape=jax.ShapeDtypeStruct((B, 1), jnp.float32),
      compiler_params=pltpu.CompilerParams(
          dimension_semantics=("parallel",)),
  )(fa, fb, nd_a, nd_b, w1fa, w1fb, wnd_a, wnd_b, b1, w2, b2, w3, b3)
  return out


def kernel(team_a_ids, team_b_ids, team_a_numerical, team_b_numerical,
           team_a_class_ids, team_b_class_ids, team_a_damage_one_hot,
           team_b_damage_one_hot, emb_table, class_table, W1, b1, W2, b2,
           W3, b3):
  tflat = lambda ids: ids.astype(jnp.int32).T
  fa, fb = _sc_pool(emb_table, class_table,
                    tflat(team_a_ids), tflat(team_a_class_ids),
                    tflat(team_b_ids), tflat(team_b_class_ids))

  nd_a = jnp.concatenate([team_a_numerical.reshape(B, 160),
                          team_a_damage_one_hot.reshape(B, 15)], axis=1)
  nd_b = jnp.concatenate([team_b_numerical.reshape(B, 160),
                          team_b_damage_one_hot.reshape(B, 15)], axis=1)

  w1fa = (jnp.concatenate([W1[0:64], W1[96:112]], 0) * INV_L).astype(BF)
  w1fb = (jnp.concatenate([W1[115:179], W1[211:227]], 0) * INV_L).astype(BF)
  wnd_a = (jnp.concatenate([jnp.tile(W1[64:96], (5, 1)),
                            jnp.tile(W1[112:115], (5, 1))], 0)
           * INV_L).astype(BF)
  wnd_b = (jnp.concatenate([jnp.tile(W1[179:211], (5, 1)),
                            jnp.tile(W1[227:230], (5, 1))], 0)
           * INV_L).astype(BF)

  out = _tc_mlp(fa, fb, nd_a, nd_b, w1fa, w1fb, wnd_a, wnd_b,
                b1.reshape(1, 512), W2.astype(BF), b2.reshape(1, 256),
                W3.astype(BF), b3.reshape(1, 1))
  return out.reshape(B)
